# baseline (speedup 1.0000x reference)
import dataclasses
import functools

import jax
import jax.numpy as jnp
from jax import lax
from jax.experimental import pallas as pl
from jax.experimental.pallas import tpu as pltpu
from jax.experimental.pallas import tpu_sc as plsc

_VOCAB = 1000000
_B = 16384
_L = 50
_D = 32
_NC = 2
_NS = 16
_NW = _NC * _NS
_BPW = _B // _NW
_C = 16
_CHUNKS = _BPW // _C
_CI = _C * _L
_G = 80
_NG = _CI // _G
_INV = 1.0 / _L


_TQ = 4
_VQ = 251904
_TW = 2048
_TSTEPS = _VQ // _TW
_VPAD = _TQ * _VQ
_LASTBLK = (_VOCAB + _TW - 1) // _TW - 1


def _transpose_tc(t_cm):
    def body(x0, x1, x2, x3, o_ref):
        for q, x in enumerate((x0, x1, x2, x3)):
            o_ref[:, q * _D:(q + 1) * _D] = x[...].astype(jnp.bfloat16).T

    def in_map(q):
        def f(i):
            return (jnp.int32(0),
                    jnp.minimum(q * _TSTEPS + i, _LASTBLK).astype(jnp.int32))
        return f

    return pl.pallas_call(
        body,
        grid=(_TSTEPS,),
        in_specs=[pl.BlockSpec((_D, _TW), in_map(q)) for q in range(_TQ)],
        out_specs=pl.BlockSpec((_TW, _TQ * _D),
                               lambda i: (i, jnp.int32(0))),
        out_shape=jax.ShapeDtypeStruct((_VQ, _TQ * _D), jnp.bfloat16),
        compiler_params=pltpu.CompilerParams(
            dimension_semantics=("arbitrary",),
        ),
    )(t_cm, t_cm, t_cm, t_cm)


def _sc_compiler_params():
    cp = pltpu.CompilerParams(use_tc_tiling_on_sc=False)
    if "needs_layout_passes" in pltpu.CompilerParams.__dataclass_fields__:
        cp = dataclasses.replace(cp, needs_layout_passes=False)
    return cp


def _cbow_sc(idx_flat, table):
    mesh = plsc.VectorSubcoreMesh(core_axis_name="c", subcore_axis_name="s")

    @functools.partial(
        pl.kernel,
        out_type=jax.ShapeDtypeStruct((_B, _D), jnp.float32),
        mesh=mesh,
        scratch_types=[
            pltpu.VMEM((_CI,), jnp.int32),
            pltpu.VMEM((_CI,), jnp.int32),
            pltpu.VMEM((_CI, _D), jnp.bfloat16),
            pltpu.VMEM((_CI, _D), jnp.bfloat16),
            pltpu.VMEM((_C, _D), jnp.float32),
            pltpu.VMEM((_C, _D), jnp.float32),
            pltpu.SemaphoreType.DMA,
            pltpu.SemaphoreType.DMA,
        ],
        compiler_params=_sc_compiler_params(),
    )
    def k(table_hbm, idx_hbm, out_hbm,
          idx0, idx1, rows0, rows1, outv0, outv1, sem0, sem1):
        wid = lax.axis_index("s") * _NC + lax.axis_index("c")
        base = wid * _BPW
        idx_b = (idx0, idx1)
        rows_b = (rows0, rows1)
        out_b = (outv0, outv1)
        sem_b = (sem0, sem1)

        def remap(idx_v):
            @pl.loop(jnp.int32(0), jnp.int32(_CI // 16))
            def _(i):
                sl = pl.ds(i * 16, 16)
                v = idx_v[sl]
                one = jnp.ones((16,), jnp.int32)
                zero = jnp.zeros((16,), jnp.int32)
                q = jnp.where(v >= _VQ, one, zero)
                q = q + jnp.where(v >= 2 * _VQ, one, zero)
                q = q + jnp.where(v >= 3 * _VQ, one, zero)
                idx_v[sl] = v * 4 - q * (4 * _VQ - 1)

        def fire(buf, c):
            row0 = base + c * _C
            pltpu.sync_copy(idx_hbm.at[pl.ds(row0 * _L, _CI)], idx_b[buf])
            remap(idx_b[buf])
            for j in range(_NG):
                pltpu.async_copy(
                    table_hbm.at[idx_b[buf].at[pl.ds(j * _G, _G)]],
                    rows_b[buf].at[pl.ds(j * _G, _G)],
                    sem_b[buf],
                )

        def drain(buf):
            pltpu.make_async_copy(
                table_hbm.at[pl.ds(0, _CI)], rows_b[buf], sem_b[buf]).wait()

        def reduce_store(buf, c):
            rows_v = rows_b[buf]
            out_v = out_b[buf]
            row0 = base + c * _C

            @pl.loop(jnp.int32(0), jnp.int32(_C))
            def _row(b):
                s = b * _L

                mask_hi = jnp.full((16,), -65536, jnp.int32)

                def split(r):
                    xi = plsc.bitcast(rows_v[r, :], jnp.int32)
                    ev = plsc.bitcast(lax.shift_left(xi, jnp.int32(16)),
                                      jnp.float32)
                    od = plsc.bitcast(jnp.bitwise_and(xi, mask_hi), jnp.float32)
                    return ev, od

                def body(i, carry):
                    r, a0, a1, b0, b1 = carry
                    ev, od = split(r)
                    a0 = a0 + ev
                    a1 = a1 + od
                    ev2, od2 = split(r + 1)
                    b0 = b0 + ev2
                    b1 = b1 + od2
                    return (r + 2, a0, a1, b0, b1)

                z = jnp.zeros((16,), jnp.float32)
                _, a0, a1, b0, b1 = lax.fori_loop(
                    0, _L // 2, body, (s, z, z, z, z), unroll=5)
                out_v[b, pl.ds(0, 16)] = (a0 + b0) * _INV
                out_v[b, pl.ds(16, 16)] = (a1 + b1) * _INV

            pltpu.sync_copy(out_v, out_hbm.at[pl.ds(row0, _C)])

        fire(0, jnp.int32(0))

        @pl.loop(jnp.int32(0), jnp.int32(_CHUNKS - 2), step=2)
        def _chunk(c):
            fire(1, c + 1)
            drain(0)
            reduce_store(0, c)
            fire(0, c + 2)
            drain(1)
            reduce_store(1, c + 1)

        fire(1, jnp.int32(_CHUNKS - 1))
        drain(0)
        reduce_store(0, jnp.int32(_CHUNKS - 2))
        drain(1)
        reduce_store(1, jnp.int32(_CHUNKS - 1))

    return k(table, idx_flat)


def kernel(input_ids, table):
    idx_flat = input_ids.reshape(-1).astype(jnp.int32)
    t_rm = _transpose_tc(table.T).reshape(_VPAD, _D)
    out = _cbow_sc(idx_flat, t_rm)
    perm = jnp.arange(_D) // 2 + (jnp.arange(_D) % 2) * 16
    return jnp.take(out, perm, axis=1)


# speedup vs baseline: 1.3166x; 1.3166x over previous
import functools

import jax
import jax.numpy as jnp
from jax import lax
from jax.experimental import pallas as pl
from jax.experimental.pallas import tpu as pltpu
from jax.experimental.pallas import tpu_sc as plsc

_VOCAB = 1000000
_B = 16384
_L = 50
_D = 32
_NC = 2
_NS = 16
_NW = _NC * _NS
_BPW = _B // _NW
_C = 16
_CHUNKS = _BPW // _C
_CI = _C * _L
_G = 80
_NG = _CI // _G
_INV = 1.0 / _L


_TQ = 4
_VQ = 253952
_TW = 4096
_TSTEPS = _VQ // _TW
_VPAD = _TQ * _VQ
_LASTBLK = (_VOCAB + _TW - 1) // _TW - 1


def _transpose_tc(t_cm):
    def body(x0, x1, x2, x3, o_ref):
        for q, x in enumerate((x0, x1, x2, x3)):
            o_ref[:, q * _D:(q + 1) * _D] = x[...].T

    def in_map(q):
        def f(i):
            return (jnp.int32(0),
                    jnp.minimum(q * _TSTEPS + i, _LASTBLK).astype(jnp.int32))
        return f

    return pl.pallas_call(
        body,
        grid=(_TSTEPS,),
        in_specs=[pl.BlockSpec((_D, _TW), in_map(q)) for q in range(_TQ)],
        out_specs=pl.BlockSpec((_TW, _TQ * _D),
                               lambda i: (i, jnp.int32(0))),
        out_shape=jax.ShapeDtypeStruct((_VQ, _TQ * _D), jnp.float32),
        compiler_params=pltpu.CompilerParams(
            dimension_semantics=("arbitrary",),
        ),
    )(t_cm, t_cm, t_cm, t_cm)


def _cbow_sc(idx_flat, table):
    mesh = plsc.VectorSubcoreMesh(core_axis_name="c", subcore_axis_name="s")

    @functools.partial(
        pl.kernel,
        out_type=jax.ShapeDtypeStruct((_B, _D), jnp.float32),
        mesh=mesh,
        scratch_types=[
            pltpu.VMEM((_CI,), jnp.int32),
            pltpu.VMEM((_CI,), jnp.int32),
            pltpu.VMEM((_CI, _D), jnp.float32),
            pltpu.VMEM((_CI, _D), jnp.float32),
            pltpu.VMEM((_C, _D), jnp.float32),
            pltpu.VMEM((_C, _D), jnp.float32),
            pltpu.SemaphoreType.DMA,
            pltpu.SemaphoreType.DMA,
        ],
        compiler_params=pltpu.CompilerParams(use_tc_tiling_on_sc=False),
    )
    def k(table_hbm, idx_hbm, out_hbm,
          idx0, idx1, rows0, rows1, outv0, outv1, sem0, sem1):
        wid = lax.axis_index("s") * _NC + lax.axis_index("c")
        base = wid * _BPW
        idx_b = (idx0, idx1)
        rows_b = (rows0, rows1)
        out_b = (outv0, outv1)
        sem_b = (sem0, sem1)

        def remap(idx_v):
            @pl.loop(jnp.int32(0), jnp.int32(_CI // 16))
            def _(i):
                sl = pl.ds(i * 16, 16)
                v = idx_v[sl]
                one = jnp.ones((16,), jnp.int32)
                zero = jnp.zeros((16,), jnp.int32)
                q = jnp.where(v >= _VQ, one, zero)
                q = q + jnp.where(v >= 2 * _VQ, one, zero)
                q = q + jnp.where(v >= 3 * _VQ, one, zero)
                idx_v[sl] = v * 4 - q * (4 * _VQ - 1)

        def fire(buf, c):
            row0 = base + c * _C
            pltpu.sync_copy(idx_hbm.at[pl.ds(row0 * _L, _CI)], idx_b[buf])
            remap(idx_b[buf])
            for j in range(_NG):
                pltpu.async_copy(
                    table_hbm.at[idx_b[buf].at[pl.ds(j * _G, _G)]],
                    rows_b[buf].at[pl.ds(j * _G, _G)],
                    sem_b[buf],
                )

        def drain(buf):
            pltpu.make_async_copy(
                table_hbm.at[pl.ds(0, _CI)], rows_b[buf], sem_b[buf]).wait()

        def reduce_store(buf, c):
            rows_v = rows_b[buf]
            out_v = out_b[buf]
            row0 = base + c * _C

            @pl.loop(jnp.int32(0), jnp.int32(_C))
            def _row(b):
                s = b * _L

                def body(i, carry):
                    r, a0, a1, b0, b1 = carry
                    a0 = a0 + rows_v[r, pl.ds(0, 16)]
                    a1 = a1 + rows_v[r, pl.ds(16, 16)]
                    b0 = b0 + rows_v[r + 1, pl.ds(0, 16)]
                    b1 = b1 + rows_v[r + 1, pl.ds(16, 16)]
                    return (r + 2, a0, a1, b0, b1)

                z = jnp.zeros((16,), jnp.float32)
                _, a0, a1, b0, b1 = lax.fori_loop(
                    0, _L // 2, body, (s, z, z, z, z), unroll=5)
                out_v[b, pl.ds(0, 16)] = (a0 + b0) * _INV
                out_v[b, pl.ds(16, 16)] = (a1 + b1) * _INV

            pltpu.sync_copy(out_v, out_hbm.at[pl.ds(row0, _C)])

        fire(0, jnp.int32(0))

        @pl.loop(jnp.int32(0), jnp.int32(_CHUNKS - 2), step=2)
        def _chunk(c):
            fire(1, c + 1)
            drain(0)
            reduce_store(0, c)
            fire(0, c + 2)
            drain(1)
            reduce_store(1, c + 1)

        fire(1, jnp.int32(_CHUNKS - 1))
        drain(0)
        reduce_store(0, jnp.int32(_CHUNKS - 2))
        drain(1)
        reduce_store(1, jnp.int32(_CHUNKS - 1))

    return k(table, idx_flat)


def kernel(input_ids, table):
    idx_flat = input_ids.reshape(-1).astype(jnp.int32)
    t_rm = _transpose_tc(table.T).reshape(_VPAD, _D)
    return _cbow_sc(idx_flat, t_rm)
